# Initial kernel scaffold; baseline (speedup 1.0000x reference)
import jax
import jax.numpy as jnp
from jax.experimental import pallas as pl


def kernel(user_id, age, sin_month, cos_month, view_count, click_count, gender, country, user_table, W1, b1, W2, b2):
    raise NotImplementedError("write your pallas kernel here")


# speedup vs baseline: 1.0180x; 1.0180x over previous
import functools

import jax
import jax.numpy as jnp
from jax import lax
from jax.experimental import pallas as pl
from jax.experimental.pallas import tpu as pltpu
from jax.experimental.pallas import tpu_sc as plsc

_B = 16384
_DIM = 16
_VOCAB_PAD = 1008
_NC = 2
_NS = 16
_NW = _NC * _NS
_BPW = _B // _NW
_GCHUNK = 128
_INV = 1.0 / (1.0 + 1e-6)


def _prep_body(ut_ref, w1a_ref, b1_ref, w1sg_ref, t_ref, sg_ref):
    t_ref[...] = (
        jnp.dot(ut_ref[...], w1a_ref[...], preferred_element_type=jnp.float32)
        + b1_ref[...]
    )
    rows = lax.broadcasted_iota(jnp.int32, (8, _DIM), 0)
    scale = jnp.where(rows < 5, jnp.float32(_INV), jnp.float32(1.0))
    sg_ref[...] = w1sg_ref[...] * scale


def _prep(ut, w1a, b1, w1sg):
    return pl.pallas_call(
        _prep_body,
        out_shape=[
            jax.ShapeDtypeStruct((_VOCAB_PAD, _DIM), jnp.float32),
            jax.ShapeDtypeStruct((8, _DIM), jnp.float32),
        ],
    )(ut, w1a, b1, w1sg)


def _sc_body(t_hbm, sg_hbm, wc_hbm, uid_hbm, g_hbm, c_hbm,
             age_hbm, sin_hbm, cos_hbm, vw_hbm, ck_hbm,
             h_hbm,
             uid_v, g_v, c_v, age_v, sin_v, cos_v, vw_v, ck_v,
             t_v, sg_v, wc_v, h_v):
    wid = lax.axis_index("s") * _NC + lax.axis_index("c")
    base = wid * _BPW

    pltpu.sync_copy(uid_hbm.at[pl.ds(base, _BPW)], uid_v)
    pltpu.sync_copy(g_hbm.at[pl.ds(base, _BPW)], g_v)
    pltpu.sync_copy(c_hbm.at[pl.ds(base, _BPW)], c_v)
    pltpu.sync_copy(age_hbm.at[pl.ds(base, _BPW)], age_v)
    pltpu.sync_copy(sin_hbm.at[pl.ds(base, _BPW)], sin_v)
    pltpu.sync_copy(cos_hbm.at[pl.ds(base, _BPW)], cos_v)
    pltpu.sync_copy(vw_hbm.at[pl.ds(base, _BPW)], vw_v)
    pltpu.sync_copy(ck_hbm.at[pl.ds(base, _BPW)], ck_v)
    pltpu.sync_copy(t_hbm, t_v)
    pltpu.sync_copy(sg_hbm, sg_v)
    pltpu.sync_copy(wc_hbm, wc_v)

    lane = lax.iota(jnp.int32, 16)
    ws0 = sg_v[0, :]
    ws1 = sg_v[1, :]
    ws2 = sg_v[2, :]
    ws3 = sg_v[3, :]
    ws4 = sg_v[4, :]

    def body(blk, carry):
        b16 = blk * 16
        u16 = uid_v[pl.ds(b16, 16)]
        g16 = g_v[pl.ds(b16, 16)]
        c16 = c_v[pl.ds(b16, 16)]
        a16 = age_v[pl.ds(b16, 16)]
        s16 = sin_v[pl.ds(b16, 16)]
        o16 = cos_v[pl.ds(b16, 16)]
        v16 = vw_v[pl.ds(b16, 16)]
        k16 = ck_v[pl.ds(b16, 16)]
        for j in range(16):
            acc = plsc.load_gather(t_v, [u16[j] * 16 + lane])
            acc = acc + sg_v[5 + g16[j], :] + wc_v[c16[j], :]
            acc = acc + a16[j] * ws0
            acc = acc + s16[j] * ws1
            acc = acc + o16[j] * ws2
            acc = acc + v16[j] * ws3
            acc = acc + k16[j] * ws4
            h_v[b16 + j, :] = jnp.maximum(acc, jnp.float32(0.0))
        return carry

    lax.fori_loop(0, _BPW // 16, body, 0)

    pltpu.sync_copy(h_v, h_hbm.at[pl.ds(base, _BPW)])


@functools.cache
def _sc_kernel():
  return pl.kernel(
    _sc_body,
    mesh=plsc.VectorSubcoreMesh(core_axis_name="c", subcore_axis_name="s"),
    compiler_params=pltpu.CompilerParams(needs_layout_passes=False),
    out_type=jax.ShapeDtypeStruct((_B, _DIM), jnp.float32),
    scratch_types=[
        pltpu.VMEM((_BPW,), jnp.int32),
        pltpu.VMEM((_BPW,), jnp.int32),
        pltpu.VMEM((_BPW,), jnp.int32),
        pltpu.VMEM((_BPW,), jnp.float32),
        pltpu.VMEM((_BPW,), jnp.float32),
        pltpu.VMEM((_BPW,), jnp.float32),
        pltpu.VMEM((_BPW,), jnp.float32),
        pltpu.VMEM((_BPW,), jnp.float32),
        pltpu.VMEM((_VOCAB_PAD * _DIM,), jnp.float32),
        pltpu.VMEM((8, _DIM), jnp.float32),
        pltpu.VMEM((32, _DIM), jnp.float32),
        pltpu.VMEM((_BPW, _DIM), jnp.float32),
    ],
  )


def _mlp2_body(h_ref, w2_ref, b2_ref, o_ref):
    o_ref[...] = (
        jnp.dot(h_ref[...], w2_ref[...], preferred_element_type=jnp.float32)
        + b2_ref[...]
    )


def _mlp2(h, w2, b2):
    return pl.pallas_call(
        _mlp2_body,
        out_shape=jax.ShapeDtypeStruct((_B, _DIM), jnp.float32),
    )(h, w2, b2)


def kernel(user_id, age, sin_month, cos_month, view_count, click_count,
           gender, country, user_table, W1, b1, W2, b2):
    ut = jnp.pad(user_table, ((0, _VOCAB_PAD - user_table.shape[0]), (0, 0)))
    t, sg = _prep(ut, W1[:_DIM], b1.reshape(1, _DIM), W1[_DIM:_DIM + 8])
    wc = W1[_DIM + 8:]
    h = _sc_kernel()(
        t.reshape(-1), sg, wc,
        user_id.astype(jnp.int32), gender.astype(jnp.int32),
        country.astype(jnp.int32),
        age, sin_month, cos_month, view_count, click_count,
    )
    return _mlp2(h, W2, b2.reshape(1, _DIM))
